# baseline (speedup 1.0000x reference)
import functools

import jax
import jax.numpy as jnp
from jax import lax
from jax.experimental import pallas as pl
from jax.experimental.pallas import tpu as pltpu
from jax.experimental.pallas import tpu_sc as plsc

T = 2048
D = 768
G = 3
E = 8
FF = 3072
WPAD = 128
BLK = 256
NBMAX = T // BLK
TP = T + (G - 1) * BLK
NBP = TP // BLK
NWORK = 32
CHUNK = T // NWORK


def _routing_body(x_ref, wg_ref, bg_ref, wcx_ref, wcp_ref, bc_ref, wr_ref,
                  br_ref, w_ref, pos_ref, meta_ref):
    x = x_ref[...]
    gl = jnp.dot(x, wg_ref[...], preferred_element_type=jnp.float32) + bg_ref[...]
    g0, g1, g2 = gl[:, 0:1], gl[:, 1:2], gl[:, 2:3]
    gid = jnp.where(g0 >= g1, 0, 1)
    gmx = jnp.maximum(g0, g1)
    gid = jnp.where(gmx >= g2, gid, 2)
    pooled = jnp.mean(x, axis=0, keepdims=True)
    cond = (jnp.dot(x, wcx_ref[...], preferred_element_type=jnp.float32)
            + jnp.dot(pooled, wcp_ref[...], preferred_element_type=jnp.float32)
            + bc_ref[...])
    el = jnp.dot(cond, wr_ref[...], preferred_element_type=jnp.float32) + br_ref[...]
    lane = lax.broadcasted_iota(jnp.int32, (1, WPAD), 1)
    seg = lane // E
    p = jnp.exp(el - jnp.max(el, axis=1, keepdims=True))
    ji = lax.broadcasted_iota(jnp.int32, (WPAD, G), 0)
    gi = lax.broadcasted_iota(jnp.int32, (WPAD, G), 1)
    S = ((ji // E == gi) & (ji < G * E)).astype(jnp.float32)
    ji2 = lax.broadcasted_iota(jnp.int32, (G, WPAD), 0)
    gi2 = lax.broadcasted_iota(jnp.int32, (G, WPAD), 1)
    S2 = ((gi2 // E == ji2) & (gi2 < G * E)).astype(jnp.float32)
    den = jnp.dot(jnp.dot(p, S, preferred_element_type=jnp.float32), S2,
                  preferred_element_type=jnp.float32)
    w = jnp.where(seg == gid, p / den, 0.0)
    w_ref[...] = w
    lane3 = lax.broadcasted_iota(jnp.int32, (1, G), 1)
    onehot = (lane3 == gid).astype(jnp.float32)
    ri = lax.broadcasted_iota(jnp.int32, (T, T), 0)
    ci = lax.broadcasted_iota(jnp.int32, (T, T), 1)
    ltri = (ri >= ci).astype(jnp.float32)
    csum = jnp.dot(ltri, onehot, preferred_element_type=jnp.float32)
    cnt = csum[T - 1:T, :]
    cap = jnp.ceil(cnt / BLK) * BLK
    starts = jnp.concatenate(
        [jnp.zeros((1, 1), jnp.float32), cap[:, 0:1], cap[:, 0:1] + cap[:, 1:2]],
        axis=1)
    pos3 = starts + csum - 1.0
    pos = jnp.sum(onehot * pos3, axis=1, keepdims=True)
    pos_ref[...] = pos.astype(jnp.int32)
    lane128 = lax.broadcasted_iota(jnp.int32, (1, 128), 1)
    meta = (jnp.where(lane128 == 0, cnt[:, 0:1], 0.0)
            + jnp.where(lane128 == 1, cnt[:, 1:2], 0.0)
            + jnp.where(lane128 == 2, cnt[:, 2:3], 0.0))
    meta_ref[...] = meta.astype(jnp.int32)


def _routing(x, Wg, bg2, Wcx, Wcp, bc2, Wr32, br32):
    return pl.pallas_call(
        _routing_body,
        out_shape=(
            jax.ShapeDtypeStruct((T, WPAD), jnp.float32),
            jax.ShapeDtypeStruct((T, 1), jnp.int32),
            jax.ShapeDtypeStruct((1, 128), jnp.int32),
        ),
    )(x, Wg, bg2, Wcx, Wcp, bc2, Wr32, br32)


@functools.cache
def _sc_kernels():
    mesh = plsc.VectorSubcoreMesh(core_axis_name="c", subcore_axis_name="s")

    @functools.partial(
        pl.kernel,
        mesh=mesh,
        out_type=(
            jax.ShapeDtypeStruct((TP, D), jnp.float32),
            jax.ShapeDtypeStruct((TP, WPAD), jnp.float32),
        ),
        scratch_types=[
            pltpu.VMEM((CHUNK,), jnp.int32),
            pltpu.VMEM((CHUNK, D), jnp.float32),
            pltpu.VMEM((CHUNK, WPAD), jnp.float32),
            pltpu.SemaphoreType.DMA,
            pltpu.SemaphoreType.DMA,
        ],
    )
    def _sc_scatter(x_hbm, w_hbm, pos_hbm, xs_hbm, ws_hbm,
                    idx_v, xrow_v, wrow_v, sem1, sem2):
        wid = lax.axis_index("s") * 2 + lax.axis_index("c")
        base = wid * CHUNK
        pltpu.sync_copy(pos_hbm.at[pl.ds(base, CHUNK)], idx_v)
        pltpu.sync_copy(x_hbm.at[pl.ds(base, CHUNK)], xrow_v)
        pltpu.sync_copy(w_hbm.at[pl.ds(base, CHUNK)], wrow_v)
        c1 = pltpu.async_copy(xrow_v, xs_hbm.at[idx_v], sem1)
        c2 = pltpu.async_copy(wrow_v, ws_hbm.at[idx_v], sem2)
        c1.wait()
        c2.wait()

    @functools.partial(
        pl.kernel,
        mesh=mesh,
        out_type=jax.ShapeDtypeStruct((T, D), jnp.float32),
        scratch_types=[
            pltpu.VMEM((CHUNK,), jnp.int32),
            pltpu.VMEM((CHUNK, D), jnp.float32),
            pltpu.SemaphoreType.DMA,
        ],
    )
    def _sc_gather(ys_hbm, pos_hbm, y_hbm, idx_v, row_v, sem):
        wid = lax.axis_index("s") * 2 + lax.axis_index("c")
        base = wid * CHUNK
        pltpu.sync_copy(pos_hbm.at[pl.ds(base, CHUNK)], idx_v)
        pltpu.async_copy(ys_hbm.at[idx_v], row_v, sem).wait()
        pltpu.sync_copy(row_v, y_hbm.at[pl.ds(base, CHUNK)])

    return _sc_scatter, _sc_gather


def _blk_of(k, j, nblk, psb):
    g = k // E
    jc = jnp.minimum(j, jnp.maximum(nblk[g] - 1, 0))
    return jnp.clip(psb[g] + jc, 0, NBP - 1)


def _ffn_body(nblk_ref, psb_ref, xs_ref, ws_ref, w1_ref, b1_ref,
              w2_ref, b2_ref, out_ref):
    k = pl.program_id(0)
    j = pl.program_id(1)
    g = k // E
    e = k % E
    nb = nblk_ref[g]
    valid = j < nb
    row = _blk_of(k, j, nblk_ref, psb_ref) * BLK
    x = xs_ref[...]
    h = jax.nn.gelu(
        jnp.dot(x, w1_ref[0, 0], preferred_element_type=jnp.float32)
        + b1_ref[0, 0])
    lane = lax.broadcasted_iota(jnp.int32, (1, WPAD), 1)
    wcol = jnp.sum(jnp.where(lane == k, ws_ref[...], 0.0),
                   axis=1, keepdims=True)
    contrib = (jnp.dot(h * wcol, w2_ref[0, 0], preferred_element_type=jnp.float32)
               + wcol * b2_ref[0, 0])

    @pl.when(valid & (e == 0))
    def _():
        out_ref[pl.ds(row, BLK), :] = x + contrib

    @pl.when(valid & (e != 0))
    def _():
        out_ref[pl.ds(row, BLK), :] = out_ref[pl.ds(row, BLK), :] + contrib


def _ffn(xs, ws, W1, b1, W2, b2, nblk, psb):
    b1 = b1.reshape(G * E, 1, FF)
    b2 = b2.reshape(G * E, 1, D)
    grid_spec = pltpu.PrefetchScalarGridSpec(
        num_scalar_prefetch=2,
        grid=(G * E, NBMAX),
        in_specs=[
            pl.BlockSpec((BLK, D), lambda k, j, nblk, psb: (_blk_of(k, j, nblk, psb), 0)),
            pl.BlockSpec((BLK, WPAD), lambda k, j, nblk, psb: (_blk_of(k, j, nblk, psb), 0)),
            pl.BlockSpec((1, 1, D, FF), lambda k, j, nblk, psb: (k // E, k % E, 0, 0)),
            pl.BlockSpec((1, 1, FF), lambda k, j, nblk, psb: (k, 0, 0)),
            pl.BlockSpec((1, 1, FF, D), lambda k, j, nblk, psb: (k // E, k % E, 0, 0)),
            pl.BlockSpec((1, 1, D), lambda k, j, nblk, psb: (k, 0, 0)),
        ],
        out_specs=pl.BlockSpec((TP, D), lambda k, j, nblk, psb: (0, 0)),
    )
    return pl.pallas_call(
        _ffn_body,
        grid_spec=grid_spec,
        out_shape=jax.ShapeDtypeStruct((TP, D), jnp.float32),
        compiler_params=pltpu.CompilerParams(
            dimension_semantics=("arbitrary", "arbitrary")),
    )(nblk, psb, xs, ws, W1, b1, W2, b2)


def kernel(x, Wg, bg, Wc, bc, Wr, br, W1, b1, W2, b2):
    Wcx = Wc[:D]
    Wcp = Wc[D:]
    Wr32 = jnp.pad(jnp.transpose(Wr, (1, 0, 2)).reshape(D, G * E),
                   ((0, 0), (0, WPAD - G * E)))
    br32 = jnp.pad(br.reshape(1, G * E), ((0, 0), (0, WPAD - G * E)))
    w32, pos2d, meta = _routing(x, Wg, bg.reshape(1, G), Wcx, Wcp,
                                bc.reshape(1, D), Wr32, br32)
    pos = pos2d.reshape(T)
    c = meta[0]
    nblk = (c[:G] + (BLK - 1)) // BLK
    psb = jnp.concatenate([jnp.zeros((1,), jnp.int32),
                           jnp.cumsum(nblk)[:-1].astype(jnp.int32)])
    sc_scatter, sc_gather = _sc_kernels()
    xs, ws = sc_scatter(x, w32, pos)
    ys = _ffn(xs, ws, W1, b1, W2, b2, nblk.astype(jnp.int32), psb)
    return sc_gather(ys, pos)


# speedup vs baseline: 1.6525x; 1.6525x over previous
import functools

import jax
import jax.numpy as jnp
from jax import lax
from jax.experimental import pallas as pl
from jax.experimental.pallas import tpu as pltpu
from jax.experimental.pallas import tpu_sc as plsc

T = 2048
D = 768
G = 3
E = 8
FF = 3072
WPAD = 128
BLK = 256
NBMAX = T // BLK
TP = T + (G - 1) * BLK
NBP = TP // BLK
NWORK = 32
CHUNK = T // NWORK


def _routing_body(x_ref, wg_ref, bg_ref, wcx_ref, wcp_ref, bc_ref, wr_ref,
                  br_ref, w_ref, pos_ref, meta_ref):
    x = x_ref[...]
    gl = jnp.dot(x, wg_ref[...], preferred_element_type=jnp.float32) + bg_ref[...]
    g0, g1, g2 = gl[:, 0:1], gl[:, 1:2], gl[:, 2:3]
    gid = jnp.where(g0 >= g1, 0, 1)
    gmx = jnp.maximum(g0, g1)
    gid = jnp.where(gmx >= g2, gid, 2)
    pooled = jnp.mean(x, axis=0, keepdims=True)
    cond = (jnp.dot(x, wcx_ref[...], preferred_element_type=jnp.float32)
            + jnp.dot(pooled, wcp_ref[...], preferred_element_type=jnp.float32)
            + bc_ref[...])
    el = jnp.dot(cond, wr_ref[...], preferred_element_type=jnp.float32) + br_ref[...]
    lane = lax.broadcasted_iota(jnp.int32, (1, WPAD), 1)
    seg = lane // E
    p = jnp.exp(el - jnp.max(el, axis=1, keepdims=True))
    ji = lax.broadcasted_iota(jnp.int32, (WPAD, G), 0)
    gi = lax.broadcasted_iota(jnp.int32, (WPAD, G), 1)
    S = ((ji // E == gi) & (ji < G * E)).astype(jnp.float32)
    ji2 = lax.broadcasted_iota(jnp.int32, (G, WPAD), 0)
    gi2 = lax.broadcasted_iota(jnp.int32, (G, WPAD), 1)
    S2 = ((gi2 // E == ji2) & (gi2 < G * E)).astype(jnp.float32)
    den = jnp.dot(jnp.dot(p, S, preferred_element_type=jnp.float32), S2,
                  preferred_element_type=jnp.float32)
    w = jnp.where(seg == gid, p / den, 0.0)
    w_ref[...] = w
    lane3 = lax.broadcasted_iota(jnp.int32, (1, G), 1)
    onehot = (lane3 == gid).astype(jnp.float32)
    ri = lax.broadcasted_iota(jnp.int32, (T, T), 0)
    ci = lax.broadcasted_iota(jnp.int32, (T, T), 1)
    ltri = (ri >= ci).astype(jnp.float32)
    csum = jnp.dot(ltri, onehot, preferred_element_type=jnp.float32)
    cnt = csum[T - 1:T, :]
    cap = jnp.ceil(cnt / BLK) * BLK
    starts = jnp.concatenate(
        [jnp.zeros((1, 1), jnp.float32), cap[:, 0:1], cap[:, 0:1] + cap[:, 1:2]],
        axis=1)
    pos3 = starts + csum - 1.0
    pos = jnp.sum(onehot * pos3, axis=1, keepdims=True)
    pos_ref[...] = pos.astype(jnp.int32)
    lane128 = lax.broadcasted_iota(jnp.int32, (1, 128), 1)
    meta = (jnp.where(lane128 == 0, cnt[:, 0:1], 0.0)
            + jnp.where(lane128 == 1, cnt[:, 1:2], 0.0)
            + jnp.where(lane128 == 2, cnt[:, 2:3], 0.0))
    meta_ref[...] = meta.astype(jnp.int32)


def _routing(x, Wg, bg2, Wcx, Wcp, bc2, Wr32, br32):
    return pl.pallas_call(
        _routing_body,
        out_shape=(
            jax.ShapeDtypeStruct((T, WPAD), jnp.float32),
            jax.ShapeDtypeStruct((T, 1), jnp.int32),
            jax.ShapeDtypeStruct((1, 128), jnp.int32),
        ),
    )(x, Wg, bg2, Wcx, Wcp, bc2, Wr32, br32)


@functools.cache
def _sc_kernels():
    mesh = plsc.VectorSubcoreMesh(core_axis_name="c", subcore_axis_name="s")

    @functools.partial(
        pl.kernel,
        mesh=mesh,
        out_type=(
            jax.ShapeDtypeStruct((TP, D), jnp.float32),
            jax.ShapeDtypeStruct((TP, WPAD), jnp.float32),
        ),
        scratch_types=[
            pltpu.VMEM((CHUNK,), jnp.int32),
            pltpu.VMEM((CHUNK, D), jnp.float32),
            pltpu.VMEM((CHUNK, WPAD), jnp.float32),
            pltpu.SemaphoreType.DMA,
            pltpu.SemaphoreType.DMA,
        ],
    )
    def _sc_scatter(x_hbm, w_hbm, pos_hbm, xs_hbm, ws_hbm,
                    idx_v, xrow_v, wrow_v, sem1, sem2):
        wid = lax.axis_index("s") * 2 + lax.axis_index("c")
        base = wid * CHUNK
        pltpu.sync_copy(pos_hbm.at[pl.ds(base, CHUNK)], idx_v)
        pltpu.sync_copy(x_hbm.at[pl.ds(base, CHUNK)], xrow_v)
        pltpu.sync_copy(w_hbm.at[pl.ds(base, CHUNK)], wrow_v)
        c1 = pltpu.async_copy(xrow_v, xs_hbm.at[idx_v], sem1)
        c2 = pltpu.async_copy(wrow_v, ws_hbm.at[idx_v], sem2)
        c1.wait()
        c2.wait()

    @functools.partial(
        pl.kernel,
        mesh=mesh,
        out_type=jax.ShapeDtypeStruct((T, D), jnp.float32),
        scratch_types=[
            pltpu.VMEM((CHUNK,), jnp.int32),
            pltpu.VMEM((CHUNK, D), jnp.float32),
            pltpu.SemaphoreType.DMA,
        ],
    )
    def _sc_gather(ys_hbm, pos_hbm, y_hbm, idx_v, row_v, sem):
        wid = lax.axis_index("s") * 2 + lax.axis_index("c")
        base = wid * CHUNK
        pltpu.sync_copy(pos_hbm.at[pl.ds(base, CHUNK)], idx_v)
        pltpu.async_copy(ys_hbm.at[idx_v], row_v, sem).wait()
        pltpu.sync_copy(row_v, y_hbm.at[pl.ds(base, CHUNK)])

    return _sc_scatter, _sc_gather


def _blk_of(k, j, nblk, psb):
    g = k // E
    jc = jnp.minimum(j, jnp.maximum(nblk[g] - 1, 0))
    return jnp.clip(psb[g] + jc, 0, NBP - 1)


def _ffn_body(nblk_ref, psb_ref, xs_ref, ws_ref, w1_ref, b1_ref,
              w2_ref, b2_ref, out_ref):
    k = pl.program_id(0)
    j = pl.program_id(1)
    g = k // E
    e = k % E
    nb = nblk_ref[g]
    valid = j < nb
    row = _blk_of(k, j, nblk_ref, psb_ref) * BLK

    @pl.when(valid)
    def _():
        x = xs_ref[...]
        h = jax.nn.gelu(
            jnp.dot(x, w1_ref[0, 0], preferred_element_type=jnp.float32)
            + b1_ref[0, 0])
        lane = lax.broadcasted_iota(jnp.int32, (1, WPAD), 1)
        wcol = jnp.sum(jnp.where(lane == k, ws_ref[...], 0.0),
                       axis=1, keepdims=True)
        contrib = wcol * (
            jnp.dot(h, w2_ref[0, 0], preferred_element_type=jnp.float32)
            + b2_ref[0, 0])

        @pl.when(e == 0)
        def _():
            out_ref[pl.ds(row, BLK), :] = x + contrib

        @pl.when(e != 0)
        def _():
            out_ref[pl.ds(row, BLK), :] = out_ref[pl.ds(row, BLK), :] + contrib


def _ffn(xs, ws, W1, b1, W2, b2, nblk, psb):
    b1 = b1.reshape(G * E, 1, FF)
    b2 = b2.reshape(G * E, 1, D)
    grid_spec = pltpu.PrefetchScalarGridSpec(
        num_scalar_prefetch=2,
        grid=(G * E, NBMAX),
        in_specs=[
            pl.BlockSpec((BLK, D), lambda k, j, nblk, psb: (_blk_of(k, j, nblk, psb), 0)),
            pl.BlockSpec((BLK, WPAD), lambda k, j, nblk, psb: (_blk_of(k, j, nblk, psb), 0)),
            pl.BlockSpec((1, 1, D, FF), lambda k, j, nblk, psb: (k // E, k % E, 0, 0)),
            pl.BlockSpec((1, 1, FF), lambda k, j, nblk, psb: (k, 0, 0)),
            pl.BlockSpec((1, 1, FF, D), lambda k, j, nblk, psb: (k // E, k % E, 0, 0)),
            pl.BlockSpec((1, 1, D), lambda k, j, nblk, psb: (k, 0, 0)),
        ],
        out_specs=pl.BlockSpec((TP, D), lambda k, j, nblk, psb: (0, 0)),
    )
    return pl.pallas_call(
        _ffn_body,
        grid_spec=grid_spec,
        out_shape=jax.ShapeDtypeStruct((TP, D), jnp.float32),
        compiler_params=pltpu.CompilerParams(
            dimension_semantics=("arbitrary", "arbitrary")),
    )(nblk, psb, xs, ws, W1, b1, W2, b2)


def kernel(x, Wg, bg, Wc, bc, Wr, br, W1, b1, W2, b2):
    Wcx = Wc[:D]
    Wcp = Wc[D:]
    Wr32 = jnp.pad(jnp.transpose(Wr, (1, 0, 2)).reshape(D, G * E),
                   ((0, 0), (0, WPAD - G * E)))
    br32 = jnp.pad(br.reshape(1, G * E), ((0, 0), (0, WPAD - G * E)))
    w32, pos2d, meta = _routing(x, Wg, bg.reshape(1, G), Wcx, Wcp,
                                bc.reshape(1, D), Wr32, br32)
    pos = pos2d.reshape(T)
    c = meta[0]
    nblk = (c[:G] + (BLK - 1)) // BLK
    psb = jnp.concatenate([jnp.zeros((1,), jnp.int32),
                           jnp.cumsum(nblk)[:-1].astype(jnp.int32)])
    sc_scatter, sc_gather = _sc_kernels()
    xs, ws = sc_scatter(x, w32, pos)
    ys = _ffn(xs, ws, W1, b1, W2, b2, nblk.astype(jnp.int32), psb)
    return sc_gather(ys, pos)


# speedup vs baseline: 1.8506x; 1.1199x over previous
import functools

import jax
import jax.numpy as jnp
from jax import lax
from jax.experimental import pallas as pl
from jax.experimental.pallas import tpu as pltpu
from jax.experimental.pallas import tpu_sc as plsc

T = 2048
D = 768
G = 3
E = 8
FF = 3072
WPAD = 128
BLK = 256
NBMAX = T // BLK
TP = T + (G - 1) * BLK
NBP = TP // BLK
NWORK = 32
CHUNK = T // NWORK


def _routing_body(x_ref, wg_ref, bg_ref, wcx_ref, wcp_ref, bc_ref, wr_ref,
                  br_ref, w_ref, pos_ref, meta_ref):
    x = x_ref[...]
    gl = jnp.dot(x, wg_ref[...], preferred_element_type=jnp.float32) + bg_ref[...]
    g0, g1, g2 = gl[:, 0:1], gl[:, 1:2], gl[:, 2:3]
    gid = jnp.where(g0 >= g1, 0, 1)
    gmx = jnp.maximum(g0, g1)
    gid = jnp.where(gmx >= g2, gid, 2)
    pooled = jnp.mean(x, axis=0, keepdims=True)
    cond = (jnp.dot(x, wcx_ref[...], preferred_element_type=jnp.float32)
            + jnp.dot(pooled, wcp_ref[...], preferred_element_type=jnp.float32)
            + bc_ref[...])
    el = jnp.dot(cond, wr_ref[...], preferred_element_type=jnp.float32) + br_ref[...]
    lane = lax.broadcasted_iota(jnp.int32, (1, WPAD), 1)
    seg = lane // E
    p = jnp.exp(el - jnp.max(el, axis=1, keepdims=True))
    ji = lax.broadcasted_iota(jnp.int32, (WPAD, G), 0)
    gi = lax.broadcasted_iota(jnp.int32, (WPAD, G), 1)
    S = ((ji // E == gi) & (ji < G * E)).astype(jnp.float32)
    ji2 = lax.broadcasted_iota(jnp.int32, (G, WPAD), 0)
    gi2 = lax.broadcasted_iota(jnp.int32, (G, WPAD), 1)
    S2 = ((gi2 // E == ji2) & (gi2 < G * E)).astype(jnp.float32)
    den = jnp.dot(jnp.dot(p, S, preferred_element_type=jnp.float32), S2,
                  preferred_element_type=jnp.float32)
    w = jnp.where(seg == gid, p / den, 0.0)
    w_ref[...] = w
    lane3 = lax.broadcasted_iota(jnp.int32, (1, G), 1)
    onehot = (lane3 == gid).astype(jnp.float32)
    ri = lax.broadcasted_iota(jnp.int32, (T, T), 0)
    ci = lax.broadcasted_iota(jnp.int32, (T, T), 1)
    ltri = (ri >= ci).astype(jnp.float32)
    csum = jnp.dot(ltri, onehot, preferred_element_type=jnp.float32)
    cnt = csum[T - 1:T, :]
    cap = jnp.ceil(cnt / BLK) * BLK
    starts = jnp.concatenate(
        [jnp.zeros((1, 1), jnp.float32), cap[:, 0:1], cap[:, 0:1] + cap[:, 1:2]],
        axis=1)
    pos3 = starts + csum - 1.0
    pos = jnp.sum(onehot * pos3, axis=1, keepdims=True)
    pos_ref[...] = pos.astype(jnp.int32)
    lane128 = lax.broadcasted_iota(jnp.int32, (1, 128), 1)
    meta = (jnp.where(lane128 == 0, cnt[:, 0:1], 0.0)
            + jnp.where(lane128 == 1, cnt[:, 1:2], 0.0)
            + jnp.where(lane128 == 2, cnt[:, 2:3], 0.0))
    meta_ref[...] = meta.astype(jnp.int32)


def _routing(x, Wg, bg2, Wcx, Wcp, bc2, Wr32, br32):
    return pl.pallas_call(
        _routing_body,
        out_shape=(
            jax.ShapeDtypeStruct((T, WPAD), jnp.float32),
            jax.ShapeDtypeStruct((T, 1), jnp.int32),
            jax.ShapeDtypeStruct((1, 128), jnp.int32),
        ),
    )(x, Wg, bg2, Wcx, Wcp, bc2, Wr32, br32)


@functools.cache
def _sc_kernels():
    mesh = plsc.VectorSubcoreMesh(core_axis_name="c", subcore_axis_name="s")

    @functools.partial(
        pl.kernel,
        mesh=mesh,
        out_type=(
            jax.ShapeDtypeStruct((TP, D), jnp.float32),
            jax.ShapeDtypeStruct((TP, WPAD), jnp.float32),
        ),
        scratch_types=[
            pltpu.VMEM((CHUNK,), jnp.int32),
            pltpu.VMEM((CHUNK, D), jnp.float32),
            pltpu.VMEM((CHUNK, WPAD), jnp.float32),
            pltpu.SemaphoreType.DMA,
            pltpu.SemaphoreType.DMA,
        ],
    )
    def _sc_scatter(x_hbm, w_hbm, pos_hbm, xs_hbm, ws_hbm,
                    idx_v, xrow_v, wrow_v, sem1, sem2):
        wid = lax.axis_index("s") * 2 + lax.axis_index("c")
        base = wid * CHUNK
        pltpu.sync_copy(pos_hbm.at[pl.ds(base, CHUNK)], idx_v)
        pltpu.sync_copy(x_hbm.at[pl.ds(base, CHUNK)], xrow_v)
        pltpu.sync_copy(w_hbm.at[pl.ds(base, CHUNK)], wrow_v)
        c1 = pltpu.async_copy(xrow_v, xs_hbm.at[idx_v], sem1)
        c2 = pltpu.async_copy(wrow_v, ws_hbm.at[idx_v], sem2)
        c1.wait()
        c2.wait()

    @functools.partial(
        pl.kernel,
        mesh=mesh,
        out_type=jax.ShapeDtypeStruct((T, D), jnp.float32),
        scratch_types=[
            pltpu.VMEM((CHUNK,), jnp.int32),
            pltpu.VMEM((CHUNK, D), jnp.float32),
            pltpu.SemaphoreType.DMA,
        ],
    )
    def _sc_gather(ys_hbm, pos_hbm, y_hbm, idx_v, row_v, sem):
        wid = lax.axis_index("s") * 2 + lax.axis_index("c")
        base = wid * CHUNK
        pltpu.sync_copy(pos_hbm.at[pl.ds(base, CHUNK)], idx_v)
        pltpu.async_copy(ys_hbm.at[idx_v], row_v, sem).wait()
        pltpu.sync_copy(row_v, y_hbm.at[pl.ds(base, CHUNK)])

    return _sc_scatter, _sc_gather


def _blk_of(k, j, nblk, psb):
    g = k // E
    jj = (NBMAX - 1) - j
    jc = jnp.minimum(jj, jnp.maximum(nblk[g] - 1, 0))
    return jnp.clip(psb[g] + jc, 0, NBP - 1)


def _ffn_body(nblk_ref, psb_ref, xs_ref, ws_ref, w1_ref, b1_ref,
              w2_ref, b2_ref, out_ref, w1b_ref, w2b_ref):
    k = pl.program_id(0)
    j = pl.program_id(1)
    g = k // E
    e = k % E
    nb = nblk_ref[g]
    valid = ((NBMAX - 1) - j) < nb
    row = _blk_of(k, j, nblk_ref, psb_ref) * BLK

    @pl.when(j == 0)
    def _():
        w1b_ref[...] = w1_ref[0, 0].astype(jnp.bfloat16)
        w2b_ref[...] = w2_ref[0, 0].astype(jnp.bfloat16)

    @pl.when(valid)
    def _():
        x = xs_ref[...]
        h = jax.nn.gelu(
            jnp.dot(x.astype(jnp.bfloat16), w1b_ref[...],
                    preferred_element_type=jnp.float32)
            + b1_ref[0, 0])
        lane = lax.broadcasted_iota(jnp.int32, (1, WPAD), 1)
        wcol = jnp.sum(jnp.where(lane == k, ws_ref[...], 0.0),
                       axis=1, keepdims=True)
        contrib = wcol * (
            jnp.dot(h.astype(jnp.bfloat16), w2b_ref[...],
                    preferred_element_type=jnp.float32)
            + b2_ref[0, 0])

        @pl.when(e == 0)
        def _():
            out_ref[pl.ds(row, BLK), :] = x + contrib

        @pl.when(e != 0)
        def _():
            out_ref[pl.ds(row, BLK), :] = out_ref[pl.ds(row, BLK), :] + contrib


def _ffn(xs, ws, W1, b1, W2, b2, nblk, psb):
    b1 = b1.reshape(G * E, 1, FF)
    b2 = b2.reshape(G * E, 1, D)
    grid_spec = pltpu.PrefetchScalarGridSpec(
        num_scalar_prefetch=2,
        grid=(G * E, NBMAX),
        in_specs=[
            pl.BlockSpec((BLK, D), lambda k, j, nblk, psb: (_blk_of(k, j, nblk, psb), 0)),
            pl.BlockSpec((BLK, WPAD), lambda k, j, nblk, psb: (_blk_of(k, j, nblk, psb), 0)),
            pl.BlockSpec((1, 1, D, FF), lambda k, j, nblk, psb: (k // E, k % E, 0, 0)),
            pl.BlockSpec((1, 1, FF), lambda k, j, nblk, psb: (k, 0, 0)),
            pl.BlockSpec((1, 1, FF, D), lambda k, j, nblk, psb: (k // E, k % E, 0, 0)),
            pl.BlockSpec((1, 1, D), lambda k, j, nblk, psb: (k, 0, 0)),
        ],
        out_specs=pl.BlockSpec((TP, D), lambda k, j, nblk, psb: (0, 0)),
        scratch_shapes=[
            pltpu.VMEM((D, FF), jnp.bfloat16),
            pltpu.VMEM((FF, D), jnp.bfloat16),
        ],
    )
    return pl.pallas_call(
        _ffn_body,
        grid_spec=grid_spec,
        out_shape=jax.ShapeDtypeStruct((TP, D), jnp.float32),
        compiler_params=pltpu.CompilerParams(
            dimension_semantics=("arbitrary", "arbitrary")),
    )(nblk, psb, xs, ws, W1, b1, W2, b2)


def kernel(x, Wg, bg, Wc, bc, Wr, br, W1, b1, W2, b2):
    Wcx = Wc[:D]
    Wcp = Wc[D:]
    Wr32 = jnp.pad(jnp.transpose(Wr, (1, 0, 2)).reshape(D, G * E),
                   ((0, 0), (0, WPAD - G * E)))
    br32 = jnp.pad(br.reshape(1, G * E), ((0, 0), (0, WPAD - G * E)))
    w32, pos2d, meta = _routing(x, Wg, bg.reshape(1, G), Wcx, Wcp,
                                bc.reshape(1, D), Wr32, br32)
    pos = pos2d.reshape(T)
    c = meta[0]
    nblk = (c[:G] + (BLK - 1)) // BLK
    psb = jnp.concatenate([jnp.zeros((1,), jnp.int32),
                           jnp.cumsum(nblk)[:-1].astype(jnp.int32)])
    sc_scatter, sc_gather = _sc_kernels()
    xs, ws = sc_scatter(x, w32, pos)
    ys = _ffn(xs, ws, W1, b1, W2, b2, nblk.astype(jnp.int32), psb)
    return sc_gather(ys, pos)


# speedup vs baseline: 2.1406x; 1.1567x over previous
import functools

import jax
import jax.numpy as jnp
from jax import lax
from jax.experimental import pallas as pl
from jax.experimental.pallas import tpu as pltpu
from jax.experimental.pallas import tpu_sc as plsc

T = 2048
D = 768
G = 3
E = 8
FF = 3072
WPAD = 128
BLK = 512
NB = T // BLK
M = NB + (G - 1)
NWORK = 32
CHUNK = T // NWORK


def _routing_body(x_ref, wg_ref, bg_ref, wcx_ref, wcp_ref, bc_ref, wr_ref,
                  br_ref, w_ref, pos_ref, meta_ref):
    x = x_ref[...]
    gl = jnp.dot(x, wg_ref[...], preferred_element_type=jnp.float32) + bg_ref[...]
    g0, g1, g2 = gl[:, 0:1], gl[:, 1:2], gl[:, 2:3]
    gid = jnp.where(g0 >= g1, 0, 1)
    gmx = jnp.maximum(g0, g1)
    gid = jnp.where(gmx >= g2, gid, 2)
    pooled = jnp.mean(x, axis=0, keepdims=True)
    cond = (jnp.dot(x, wcx_ref[...], preferred_element_type=jnp.float32)
            + jnp.dot(pooled, wcp_ref[...], preferred_element_type=jnp.float32)
            + bc_ref[...])
    el = jnp.dot(cond, wr_ref[...], preferred_element_type=jnp.float32) + br_ref[...]
    lane = lax.broadcasted_iota(jnp.int32, (1, WPAD), 1)
    seg = lane // E
    p = jnp.exp(el - jnp.max(el, axis=1, keepdims=True))
    ji = lax.broadcasted_iota(jnp.int32, (WPAD, G), 0)
    gi = lax.broadcasted_iota(jnp.int32, (WPAD, G), 1)
    S = ((ji // E == gi) & (ji < G * E)).astype(jnp.float32)
    ji2 = lax.broadcasted_iota(jnp.int32, (G, WPAD), 0)
    gi2 = lax.broadcasted_iota(jnp.int32, (G, WPAD), 1)
    S2 = ((gi2 // E == ji2) & (gi2 < G * E)).astype(jnp.float32)
    den = jnp.dot(jnp.dot(p, S, preferred_element_type=jnp.float32), S2,
                  preferred_element_type=jnp.float32)
    w = jnp.where(seg == gid, p / den, 0.0)
    w_ref[...] = w
    lane3 = lax.broadcasted_iota(jnp.int32, (1, G), 1)
    onehot = (lane3 == gid).astype(jnp.float32)
    ri = lax.broadcasted_iota(jnp.int32, (T, T), 0)
    ci = lax.broadcasted_iota(jnp.int32, (T, T), 1)
    ltri = (ri >= ci).astype(jnp.float32)
    csum = jnp.dot(ltri, onehot, preferred_element_type=jnp.float32)
    cnt = csum[T - 1:T, :]
    starts = jnp.concatenate(
        [jnp.zeros((1, 1), jnp.float32), cnt[:, 0:1], cnt[:, 0:1] + cnt[:, 1:2]],
        axis=1)
    pos3 = starts + csum - 1.0
    pos = jnp.sum(onehot * pos3, axis=1, keepdims=True)
    pos_ref[...] = pos.astype(jnp.int32)
    lane128 = lax.broadcasted_iota(jnp.int32, (1, 128), 1)
    meta = (jnp.where(lane128 == 0, cnt[:, 0:1], 0.0)
            + jnp.where(lane128 == 1, cnt[:, 1:2], 0.0)
            + jnp.where(lane128 == 2, cnt[:, 2:3], 0.0))
    meta_ref[...] = meta.astype(jnp.int32)


def _routing(x, Wg, bg2, Wcx, Wcp, bc2, Wr32, br32):
    return pl.pallas_call(
        _routing_body,
        out_shape=(
            jax.ShapeDtypeStruct((T, WPAD), jnp.float32),
            jax.ShapeDtypeStruct((T, 1), jnp.int32),
            jax.ShapeDtypeStruct((1, 128), jnp.int32),
        ),
    )(x, Wg, bg2, Wcx, Wcp, bc2, Wr32, br32)


@functools.cache
def _sc_kernels():
    mesh = plsc.VectorSubcoreMesh(core_axis_name="c", subcore_axis_name="s")

    @functools.partial(
        pl.kernel,
        mesh=mesh,
        out_type=(
            jax.ShapeDtypeStruct((T, D), jnp.float32),
            jax.ShapeDtypeStruct((T, WPAD), jnp.float32),
        ),
        scratch_types=[
            pltpu.VMEM((CHUNK,), jnp.int32),
            pltpu.VMEM((CHUNK, D), jnp.float32),
            pltpu.VMEM((CHUNK, WPAD), jnp.float32),
            pltpu.SemaphoreType.DMA,
            pltpu.SemaphoreType.DMA,
        ],
    )
    def _sc_scatter(x_hbm, w_hbm, pos_hbm, xs_hbm, ws_hbm,
                    idx_v, xrow_v, wrow_v, sem1, sem2):
        wid = lax.axis_index("s") * 2 + lax.axis_index("c")
        base = wid * CHUNK
        pltpu.sync_copy(pos_hbm.at[pl.ds(base, CHUNK)], idx_v)
        pltpu.sync_copy(x_hbm.at[pl.ds(base, CHUNK)], xrow_v)
        pltpu.sync_copy(w_hbm.at[pl.ds(base, CHUNK)], wrow_v)
        c1 = pltpu.async_copy(xrow_v, xs_hbm.at[idx_v], sem1)
        c2 = pltpu.async_copy(wrow_v, ws_hbm.at[idx_v], sem2)
        c1.wait()
        c2.wait()

    @functools.partial(
        pl.kernel,
        mesh=mesh,
        out_type=jax.ShapeDtypeStruct((T, D), jnp.float32),
        scratch_types=[
            pltpu.VMEM((CHUNK,), jnp.int32),
            pltpu.VMEM((CHUNK, D), jnp.float32),
            pltpu.SemaphoreType.DMA,
        ],
    )
    def _sc_gather(ys_hbm, pos_hbm, y_hbm, idx_v, row_v, sem):
        wid = lax.axis_index("s") * 2 + lax.axis_index("c")
        base = wid * CHUNK
        pltpu.sync_copy(pos_hbm.at[pl.ds(base, CHUNK)], idx_v)
        pltpu.async_copy(ys_hbm.at[idx_v], row_v, sem).wait()
        pltpu.sync_copy(row_v, y_hbm.at[pl.ds(base, CHUNK)])

    return _sc_scatter, _sc_gather


def _ffn_body(tb_ref, gi_ref, vd_ref, xs_ref, ws_ref, w1_ref, b1_ref,
              w2_ref, b2_ref, out_ref):
    i = pl.program_id(0)
    e = pl.program_id(1)
    gi = gi_ref[i]
    x = xs_ref[...]
    h = jax.nn.gelu(
        jnp.dot(x, w1_ref[0, 0], preferred_element_type=jnp.float32)
        + b1_ref[0, 0])
    lane = lax.broadcasted_iota(jnp.int32, (1, WPAD), 1)
    wcol = jnp.sum(jnp.where(lane == gi * E + e, ws_ref[...], 0.0),
                   axis=1, keepdims=True)
    wcol = wcol * vd_ref[i].astype(jnp.float32)
    contrib = wcol * (
        jnp.dot(h, w2_ref[0, 0], preferred_element_type=jnp.float32)
        + b2_ref[0, 0])
    prev = tb_ref[jnp.maximum(i - 1, 0)]
    first = (e == 0) & ((i == 0) | (tb_ref[i] != prev))

    @pl.when(first)
    def _():
        out_ref[...] = x + contrib

    @pl.when(jnp.logical_not(first))
    def _():
        out_ref[...] = out_ref[...] + contrib


def _ffn(xs, ws, W1, b1, W2, b2, tbv, giv, vdv):
    b1 = b1.reshape(G * E, 1, FF)
    b2 = b2.reshape(G * E, 1, D)
    grid_spec = pltpu.PrefetchScalarGridSpec(
        num_scalar_prefetch=3,
        grid=(M, E),
        in_specs=[
            pl.BlockSpec((BLK, D), lambda i, e, tb, gi, vd: (tb[i], 0)),
            pl.BlockSpec((BLK, WPAD), lambda i, e, tb, gi, vd: (tb[i], 0)),
            pl.BlockSpec((1, 1, D, FF), lambda i, e, tb, gi, vd: (gi[i], e, 0, 0)),
            pl.BlockSpec((1, 1, FF), lambda i, e, tb, gi, vd: (gi[i] * E + e, 0, 0)),
            pl.BlockSpec((1, 1, FF, D), lambda i, e, tb, gi, vd: (gi[i], e, 0, 0)),
            pl.BlockSpec((1, 1, D), lambda i, e, tb, gi, vd: (gi[i] * E + e, 0, 0)),
        ],
        out_specs=pl.BlockSpec((BLK, D), lambda i, e, tb, gi, vd: (tb[i], 0)),
    )
    return pl.pallas_call(
        _ffn_body,
        grid_spec=grid_spec,
        out_shape=jax.ShapeDtypeStruct((T, D), jnp.float32),
        compiler_params=pltpu.CompilerParams(
            dimension_semantics=("arbitrary", "arbitrary")),
    )(tbv, giv, vdv, xs, ws, W1, b1, W2, b2)


def kernel(x, Wg, bg, Wc, bc, Wr, br, W1, b1, W2, b2):
    Wcx = Wc[:D]
    Wcp = Wc[D:]
    Wr32 = jnp.pad(jnp.transpose(Wr, (1, 0, 2)).reshape(D, G * E),
                   ((0, 0), (0, WPAD - G * E)))
    br32 = jnp.pad(br.reshape(1, G * E), ((0, 0), (0, WPAD - G * E)))
    w32, pos2d, meta = _routing(x, Wg, bg.reshape(1, G), Wcx, Wcp,
                                bc.reshape(1, D), Wr32, br32)
    pos = pos2d.reshape(T)
    c = meta[0]
    bnd1, bnd2 = c[0], c[0] + c[1]
    bstart = jnp.arange(NB, dtype=jnp.int32) * BLK
    bend = bstart + (BLK - 1)
    gof = lambda t: (t >= bnd1).astype(jnp.int32) + (t >= bnd2).astype(jnp.int32)
    gmin = gof(bstart)
    gmax = gof(bend)
    cnt = gmax - gmin + 1
    offs = jnp.concatenate([jnp.zeros((1,), jnp.int32), jnp.cumsum(cnt)[:-1]])
    total = offs[-1] + cnt[-1]
    ii = jnp.arange(M, dtype=jnp.int32)
    jidx = jnp.clip(jnp.searchsorted(offs, ii, side="right").astype(jnp.int32) - 1,
                    0, NB - 1)
    tbv = jidx
    giv = jnp.clip(gmin[jidx] + ii - offs[jidx], 0, G - 1)
    vdv = (ii < total).astype(jnp.int32)
    sc_scatter, sc_gather = _sc_kernels()
    xs, ws = sc_scatter(x, w32, pos)
    ys = _ffn(xs, ws, W1, b1, W2, b2, tbv, giv, vdv)
    return sc_gather(ys, pos)
